# baseline (speedup 1.0000x reference)
import functools

import jax
import jax.numpy as jnp
from jax import lax
from jax.experimental import pallas as pl
from jax.experimental.pallas import tpu as pltpu
from jax.experimental.pallas import tpu_sc as plsc

S, D, NF, E, K, H = 2048, 1024, 8, 8, 2, 1024
NS = 3
EPS = 1e-05
TT = 256
NTILES = 24
NT_TOT = NTILES * TT
NPAIR = S * K
RT = S // TT
NC = 2
NW = 32
CHUNK = 128
ITERS = NPAIR // (NW * CHUNK)
LANES = 128
SL = (D // 2) // LANES


def _router_body(h_ref, a0_ref, a1_ref, gp_ref, feat_ref, lng_ref, lnb_ref,
                 wr_ref, br_ref,
                 hout_ref, hn_ref, dst_ref, eid_ref, g_ref,
                 cum1_scr, cum2_scr, o1_scr, o2_scr, c1c_scr, c2c_scr):
    i = pl.program_id(0)

    @pl.when(i == 0)
    def _init():
        c1c_scr[...] = jnp.zeros((1, E), jnp.float32)
        c2c_scr[...] = jnp.zeros((1, E), jnp.float32)

    @pl.when(i < RT)
    def _tile():
        rows = pl.ds(i * TT, TT)
        h = (h_ref[...]
             + gp_ref[:, 0:1] * a0_ref[...].astype(jnp.float32)
             + gp_ref[:, 1:2] * a1_ref[...].astype(jnp.float32))
        hout_ref[...] = h
        mu = jnp.mean(h, axis=1, keepdims=True)
        var = jnp.mean((h - mu) ** 2, axis=1, keepdims=True)
        hn = (h - mu) / jnp.sqrt(var + EPS) * lng_ref[0, :] + lnb_ref[0, :]
        hnh = hn.astype(jnp.bfloat16)
        hn_ref[...] = hnh

        logits = (jnp.dot(hnh, wr_ref[:D, :].astype(jnp.bfloat16),
                          preferred_element_type=jnp.float32)
                  + jnp.dot(feat_ref[...].astype(jnp.bfloat16),
                            wr_ref[D:, :].astype(jnp.bfloat16),
                            preferred_element_type=jnp.float32)
                  + br_ref[0, :])
        m = jnp.max(logits, axis=1, keepdims=True)
        ex = jnp.exp(logits - m)
        probs = ex / jnp.sum(ex, axis=1, keepdims=True)

        idx = lax.broadcasted_iota(jnp.int32, (TT, E), 1)
        m1 = jnp.max(probs, axis=1, keepdims=True)
        i1 = jnp.min(jnp.where(probs == m1, idx, E), axis=1, keepdims=True)
        o1 = (idx == i1).astype(jnp.float32)
        probs2 = jnp.where(idx == i1, -1.0, probs)
        m2 = jnp.max(probs2, axis=1, keepdims=True)
        i2 = jnp.min(jnp.where(probs2 == m2, idx, E), axis=1, keepdims=True)
        o2 = (idx == i2).astype(jnp.float32)
        denom = m1 + m2 + 1e-9
        g_ref[...] = jnp.concatenate([m1 / denom, m2 / denom], axis=1)

        o1_scr[rows, :] = o1
        o2_scr[rows, :] = o2
        c1 = o1
        c2 = o2
        for sh in (1, 2, 4, 8, 16, 32, 64, 128):
            z = jnp.zeros((sh, E), jnp.float32)
            c1 = c1 + jnp.concatenate([z, c1[:TT - sh, :]], axis=0)
            c2 = c2 + jnp.concatenate([z, c2[:TT - sh, :]], axis=0)
        c1 = c1 + c1c_scr[...]
        c2 = c2 + c2c_scr[...]
        cum1_scr[rows, :] = c1
        cum2_scr[rows, :] = c2
        c1c_scr[...] = c1[TT - 1:TT, :]
        c2c_scr[...] = c2[TT - 1:TT, :]

    @pl.when(i == RT)
    def _meta():
        c1 = c1c_scr[...]
        cnt = (c1 + c2c_scr[...]).astype(jnp.int32)
        nt = (cnt + (TT - 1)) // TT
        t = nt
        for sh in (1, 2, 4):
            z = jnp.zeros((1, sh), jnp.int32)
            t = t + jnp.concatenate([z, t[:, :E - sh]], axis=1)
        start = t - nt

        jrow = lax.broadcasted_iota(jnp.int32, (NTILES, E), 0)
        ind = (jnp.broadcast_to(start, (NTILES, E)) <= jrow).astype(jnp.int32)
        eid_ref[...] = jnp.sum(ind, axis=1, keepdims=True) - 1

        startf = jnp.broadcast_to(start.astype(jnp.float32) * TT, (S, E))
        c1f = jnp.broadcast_to(c1, (S, E))
        o1 = o1_scr[...]
        o2 = o2_scr[...]
        dst0 = (jnp.sum(o1 * startf, axis=1, keepdims=True)
                + jnp.sum(o1 * cum1_scr[...], axis=1, keepdims=True) - 1.0)
        dst1 = (jnp.sum(o2 * startf, axis=1, keepdims=True)
                + jnp.sum(o2 * c1f, axis=1, keepdims=True)
                + jnp.sum(o2 * cum2_scr[...], axis=1, keepdims=True) - 1.0)
        dst_ref[...] = jnp.concatenate([dst0, dst1], axis=1).astype(jnp.int32)


def _run_router(h, a, gp, feat, lng, lnb, wr, br):
    ri = lambda i: (jnp.minimum(i, RT - 1), 0)
    return pl.pallas_call(
        _router_body,
        grid=(RT + 1,),
        in_specs=[
            pl.BlockSpec((TT, D), ri),
            pl.BlockSpec((TT, D), lambda i: (jnp.minimum(i, RT - 1), 0)),
            pl.BlockSpec((TT, D), lambda i: (jnp.minimum(i, RT - 1) + RT, 0)),
            pl.BlockSpec((TT, K), ri),
            pl.BlockSpec((TT, NF), ri),
            pl.BlockSpec((1, D), lambda i: (0, 0)),
            pl.BlockSpec((1, D), lambda i: (0, 0)),
            pl.BlockSpec((D + NF, E), lambda i: (0, 0)),
            pl.BlockSpec((1, E), lambda i: (0, 0)),
        ],
        out_specs=[
            pl.BlockSpec((TT, D), ri),
            pl.BlockSpec((TT, D), ri),
            pl.BlockSpec((S, K), lambda i: (0, 0)),
            pl.BlockSpec((NTILES, 1), lambda i: (0, 0)),
            pl.BlockSpec((TT, K), ri),
        ],
        out_shape=[
            jax.ShapeDtypeStruct((S, D), jnp.float32),
            jax.ShapeDtypeStruct((S, D), jnp.bfloat16),
            jax.ShapeDtypeStruct((S, K), jnp.int32),
            jax.ShapeDtypeStruct((NTILES, 1), jnp.int32),
            jax.ShapeDtypeStruct((S, K), jnp.float32),
        ],
        scratch_shapes=[
            pltpu.VMEM((S, E), jnp.float32),
            pltpu.VMEM((S, E), jnp.float32),
            pltpu.VMEM((S, E), jnp.float32),
            pltpu.VMEM((S, E), jnp.float32),
            pltpu.VMEM((1, E), jnp.float32),
            pltpu.VMEM((1, E), jnp.float32),
        ],
        compiler_params=pltpu.CompilerParams(
            dimension_semantics=("arbitrary",),
        ),
    )(h, a, a, gp, feat, lng, lnb, wr, br)


def _scatter_body(hn_ref, dst_ref, xs_ref, idx_v, rows_v, sem):
    wid = lax.axis_index("s") * 2 + lax.axis_index("c")
    for it in range(ITERS):
        p0 = wid * (ITERS * CHUNK) + it * CHUNK
        t0 = lax.rem(p0, S)
        pltpu.sync_copy(dst_ref.at[wid, it], idx_v)
        pltpu.sync_copy(hn_ref.at[pl.ds(t0, CHUNK)], rows_v)
        pltpu.async_copy(rows_v, xs_ref.at[idx_v], sem).wait()


def _sc_scatter(hn3, dst_sc):
    call = functools.partial(
        pl.kernel,
        mesh=plsc.VectorSubcoreMesh(core_axis_name="c", subcore_axis_name="s",
                                    num_cores=2),
        out_type=jax.ShapeDtypeStruct((NT_TOT, SL, LANES), jnp.int32),
        scratch_types=[
            pltpu.VMEM((CHUNK,), jnp.int32),
            pltpu.VMEM((CHUNK, SL, LANES), jnp.int32),
            pltpu.SemaphoreType.DMA,
        ],
    )(_scatter_body)
    return call(hn3, dst_sc)


def _gather_body(ys_ref, dst_ref, a_ref, idx_v, rows_v, sem):
    wid = lax.axis_index("s") * 2 + lax.axis_index("c")
    for it in range(ITERS):
        p0 = wid * (ITERS * CHUNK) + it * CHUNK
        pltpu.sync_copy(dst_ref.at[wid, it], idx_v)
        pltpu.async_copy(ys_ref.at[idx_v], rows_v, sem).wait()
        pltpu.sync_copy(rows_v, a_ref.at[pl.ds(p0, CHUNK)])


def _sc_gather(ys3, dst_sc):
    call = functools.partial(
        pl.kernel,
        mesh=plsc.VectorSubcoreMesh(core_axis_name="c", subcore_axis_name="s",
                                    num_cores=2),
        out_type=jax.ShapeDtypeStruct((NPAIR, SL, LANES), jnp.int32),
        scratch_types=[
            pltpu.VMEM((CHUNK,), jnp.int32),
            pltpu.VMEM((CHUNK, SL, LANES), jnp.int32),
            pltpu.SemaphoreType.DMA,
        ],
    )(_gather_body)
    return call(ys3, dst_sc)


def _gemm_body(eid_ref, xs_ref, w1_ref, b1_ref, w2_ref, b2_ref, ys_ref):
    x = xs_ref[...]
    a1 = jnp.dot(x, w1_ref[0], preferred_element_type=jnp.float32) + b1_ref[0]
    eh = jax.nn.gelu(a1)
    eo = (jnp.dot(eh.astype(jnp.bfloat16), w2_ref[0],
                  preferred_element_type=jnp.float32) + b2_ref[0])
    ys_ref[...] = eo.astype(jnp.bfloat16)


def _run_gemm(eid, xs, w1, b1, w2, b2):
    tpc = NTILES // NC
    return pl.pallas_call(
        _gemm_body,
        grid_spec=pltpu.PrefetchScalarGridSpec(
            num_scalar_prefetch=1,
            grid=(NC, tpc),
            in_specs=[
                pl.BlockSpec((TT, D), lambda c, j, e_ref: (c * tpc + j, 0)),
                pl.BlockSpec((1, D, H),
                             lambda c, j, e_ref: (e_ref[c * tpc + j], 0, 0)),
                pl.BlockSpec((1, 1, H),
                             lambda c, j, e_ref: (e_ref[c * tpc + j], 0, 0)),
                pl.BlockSpec((1, H, D),
                             lambda c, j, e_ref: (e_ref[c * tpc + j], 0, 0)),
                pl.BlockSpec((1, 1, D),
                             lambda c, j, e_ref: (e_ref[c * tpc + j], 0, 0)),
            ],
            out_specs=pl.BlockSpec((TT, D), lambda c, j, e_ref: (c * tpc + j, 0)),
        ),
        out_shape=jax.ShapeDtypeStruct((NT_TOT, D), jnp.bfloat16),
        compiler_params=pltpu.CompilerParams(
            dimension_semantics=("parallel", "arbitrary"),
        ),
    )(eid, xs, w1, b1, w2, b2)


def _final_body(h_ref, a0_ref, a1_ref, g_ref, out_ref):
    out_ref[...] = (h_ref[...]
                    + g_ref[:, 0:1] * a0_ref[...].astype(jnp.float32)
                    + g_ref[:, 1:2] * a1_ref[...].astype(jnp.float32))


def _run_final(h, a, g):
    return pl.pallas_call(
        _final_body,
        grid=(RT,),
        in_specs=[
            pl.BlockSpec((TT, D), lambda i: (i, 0)),
            pl.BlockSpec((TT, D), lambda i: (i, 0)),
            pl.BlockSpec((TT, D), lambda i: (i + RT, 0)),
            pl.BlockSpec((TT, K), lambda i: (i, 0)),
        ],
        out_specs=pl.BlockSpec((TT, D), lambda i: (i, 0)),
        out_shape=jax.ShapeDtypeStruct((S, D), jnp.float32),
    )(h, a, a, g)


def _as_i32(rows):
    n = rows.shape[0]
    return lax.bitcast_convert_type(
        rows.reshape(n, D // 2, 2), jnp.int32).reshape(n, SL, LANES)


def _as_bf16(rows3):
    n = rows3.shape[0]
    return lax.bitcast_convert_type(
        rows3.reshape(n, D // 2), jnp.bfloat16).reshape(n, D)


def _dispatch(hn, dst_sc):
    return _as_bf16(_sc_scatter(_as_i32(hn), dst_sc))


def _combine(ys, dst_sc):
    return _as_bf16(_sc_gather(_as_i32(ys), dst_sc))


def kernel(hidden, feat,
           ln_g0, ln_b0, Wr0, br0, W1_0, b1_0, W2_0, b2_0,
           ln_g1, ln_b1, Wr1, br1, W1_1, b1_1, W2_1, b2_1,
           ln_g2, ln_b2, Wr2, br2, W1_2, b1_2, W2_2, b2_2):
    stage_params = [
        (ln_g0, ln_b0, Wr0, br0, W1_0, b1_0, W2_0, b2_0),
        (ln_g1, ln_b1, Wr1, br1, W1_1, b1_1, W2_1, b2_1),
        (ln_g2, ln_b2, Wr2, br2, W1_2, b1_2, W2_2, b2_2),
    ]
    h = hidden.reshape(S, D)
    ft = feat.reshape(S, NF)
    a = jnp.zeros((NPAIR, D), jnp.bfloat16)
    g = jnp.zeros((S, K), jnp.float32)
    for lng, lnb, wr, br, w1, b1, w2, b2 in stage_params:
        h, hn, dst, eid, g = _run_router(
            h, a, g, ft, lng.reshape(1, D), lnb.reshape(1, D), wr,
            br.reshape(1, E))
        dst_sc = dst.T.reshape(NW, ITERS, CHUNK)
        xs = _dispatch(hn, dst_sc)
        ys = _run_gemm(eid.reshape(NTILES), xs,
                       w1.astype(jnp.bfloat16), b1.reshape(E, 1, H),
                       w2.astype(jnp.bfloat16), b2.reshape(E, 1, D))
        a = _combine(ys, dst_sc)
    out = _run_final(h, a, g)
    return out.reshape(hidden.shape)


# speedup vs baseline: 3.4488x; 3.4488x over previous
import jax
import jax.numpy as jnp
from jax.experimental import pallas as pl
from jax.experimental.pallas import tpu as pltpu

S, D, NF, E, K, H = 2048, 1024, 8, 8, 2, 1024
NS = 3
EPS = 1e-05
NC = 2
TT = 512
SPC = S // NC
TPC = SPC // TT


def _moe_body(hid_ref, feat_ref, lng_ref, lnb_ref, wr_ref, br_ref,
              w1_ref, b1_ref, w2_ref, b2_ref, out_ref,
              h_scr, hn_scr, g_scr, acc_scr):
    s = pl.program_id(1)
    e = pl.program_id(2)
    i = pl.program_id(3)
    rows = pl.ds(i * TT, TT)

    @pl.when(e == 0)
    def _router():
        @pl.when(s == 0)
        def _():
            h_scr[rows, :] = hid_ref[rows, :]

        @pl.when(s > 0)
        def _():
            h_scr[rows, :] = h_scr[rows, :] + acc_scr[rows, :]

        h = h_scr[rows, :]
        mu = jnp.mean(h, axis=1, keepdims=True)
        var = jnp.mean((h - mu) ** 2, axis=1, keepdims=True)
        hn = (h - mu) / jnp.sqrt(var + EPS) * lng_ref[0, 0, :] + lnb_ref[0, 0, :]
        hnh = hn.astype(jnp.bfloat16)
        hn_scr[rows, :] = hnh

        logits = (jnp.dot(hnh, wr_ref[0, :D, :].astype(jnp.bfloat16),
                          preferred_element_type=jnp.float32)
                  + jnp.dot(feat_ref[rows, :].astype(jnp.bfloat16),
                            wr_ref[0, D:, :].astype(jnp.bfloat16),
                            preferred_element_type=jnp.float32)
                  + br_ref[0, 0, :])
        m = jnp.max(logits, axis=1, keepdims=True)
        ex = jnp.exp(logits - m)
        probs = ex / jnp.sum(ex, axis=1, keepdims=True)

        idx = jax.lax.broadcasted_iota(jnp.int32, (TT, E), 1)
        m1 = jnp.max(probs, axis=1, keepdims=True)
        i1 = jnp.min(jnp.where(probs == m1, idx, E), axis=1, keepdims=True)
        oh1 = idx == i1
        probs2 = jnp.where(oh1, -1.0, probs)
        m2 = jnp.max(probs2, axis=1, keepdims=True)
        i2 = jnp.min(jnp.where(probs2 == m2, idx, E), axis=1, keepdims=True)
        oh2 = idx == i2
        denom = m1 + m2 + 1e-9
        g_scr[rows, :] = (jnp.where(oh1, m1, 0.0) + jnp.where(oh2, m2, 0.0)) / denom

    idx = jax.lax.broadcasted_iota(jnp.int32, (TT, E), 1)
    ge = jnp.sum(jnp.where(idx == e, g_scr[rows, :], 0.0),
                 axis=1, keepdims=True)

    def _accumulate(contrib):
        @pl.when(e == 0)
        def _():
            acc_scr[rows, :] = contrib

        @pl.when(e > 0)
        def _():
            acc_scr[rows, :] = acc_scr[rows, :] + contrib

    hnb = hn_scr[rows, :]
    a1 = jnp.dot(hnb, w1_ref[0, 0],
                 preferred_element_type=jnp.float32) + b1_ref[0, 0, :]
    eh = jax.nn.gelu(a1)
    eo = jnp.dot(eh.astype(jnp.bfloat16), w2_ref[0, 0],
                 preferred_element_type=jnp.float32)
    _accumulate(ge * (eo + b2_ref[0, 0, :]))

    @pl.when((s == NS - 1) & (e == E - 1))
    def _():
        out_ref[rows, :] = h_scr[rows, :] + acc_scr[rows, :]


def _run_moe(hid, feat, lng, lnb, wr, br, w1, b1, w2, b2):
    grid = (NC, NS, E, TPC)
    return pl.pallas_call(
        _moe_body,
        grid=grid,
        in_specs=[
            pl.BlockSpec((SPC, D), lambda c, s, e, i: (c, 0)),
            pl.BlockSpec((SPC, NF), lambda c, s, e, i: (c, 0)),
            pl.BlockSpec((1, 1, D), lambda c, s, e, i: (s, 0, 0)),
            pl.BlockSpec((1, 1, D), lambda c, s, e, i: (s, 0, 0)),
            pl.BlockSpec((1, D + NF, E), lambda c, s, e, i: (s, 0, 0)),
            pl.BlockSpec((1, 1, E), lambda c, s, e, i: (s, 0, 0)),
            pl.BlockSpec((1, 1, D, H), lambda c, s, e, i: (s, e, 0, 0)),
            pl.BlockSpec((1, 1, H), lambda c, s, e, i: (s * E + e, 0, 0)),
            pl.BlockSpec((1, 1, H, D), lambda c, s, e, i: (s, e, 0, 0)),
            pl.BlockSpec((1, 1, D), lambda c, s, e, i: (s * E + e, 0, 0)),
        ],
        out_specs=pl.BlockSpec((SPC, D), lambda c, s, e, i: (c, 0)),
        out_shape=jax.ShapeDtypeStruct((S, D), jnp.float32),
        scratch_shapes=[
            pltpu.VMEM((SPC, D), jnp.float32),
            pltpu.VMEM((SPC, D), jnp.bfloat16),
            pltpu.VMEM((SPC, E), jnp.float32),
            pltpu.VMEM((SPC, D), jnp.float32),
        ],
        compiler_params=pltpu.CompilerParams(
            dimension_semantics=("parallel", "arbitrary", "arbitrary",
                                 "arbitrary"),
        ),
    )(hid, feat, lng, lnb, wr, br, w1, b1, w2, b2)


def kernel(hidden, feat,
           ln_g0, ln_b0, Wr0, br0, W1_0, b1_0, W2_0, b2_0,
           ln_g1, ln_b1, Wr1, br1, W1_1, b1_1, W2_1, b2_1,
           ln_g2, ln_b2, Wr2, br2, W1_2, b1_2, W2_2, b2_2):
    lng = jnp.stack([ln_g0, ln_g1, ln_g2]).reshape(NS, 1, D)
    lnb = jnp.stack([ln_b0, ln_b1, ln_b2]).reshape(NS, 1, D)
    wr = jnp.stack([Wr0, Wr1, Wr2])
    br = jnp.stack([br0, br1, br2]).reshape(NS, 1, E)
    w1 = jnp.stack([W1_0, W1_1, W1_2]).astype(jnp.bfloat16)
    b1 = jnp.stack([b1_0, b1_1, b1_2]).reshape(NS * E, 1, H)
    w2 = jnp.stack([W2_0, W2_1, W2_2]).astype(jnp.bfloat16)
    b2 = jnp.stack([b2_0, b2_1, b2_2]).reshape(NS * E, 1, D)
    out = _run_moe(hidden.reshape(S, D), feat.reshape(S, NF),
                   lng, lnb, wr, br, w1, b1, w2, b2)
    return out.reshape(hidden.shape)


# speedup vs baseline: 3.7079x; 1.0751x over previous
import jax
import jax.numpy as jnp
from jax.experimental import pallas as pl
from jax.experimental.pallas import tpu as pltpu

S, D, NF, E, K, H = 2048, 1024, 8, 8, 2, 1024
NS = 3
EPS = 1e-05
NC = 2
TT = 1024
SPC = S // NC
TPC = SPC // TT


def _moe_body(hid_ref, feat_ref, lng_ref, lnb_ref, wr_ref, br_ref,
              w1_ref, b1_ref, w2_ref, b2_ref, out_ref,
              h_scr, hn_scr, g_scr, acc_scr):
    s = pl.program_id(1)
    e = pl.program_id(2)
    i = pl.program_id(3)
    rows = pl.ds(i * TT, TT)

    @pl.when(e == 0)
    def _router():
        @pl.when(s == 0)
        def _():
            h_scr[rows, :] = hid_ref[rows, :]

        @pl.when(s > 0)
        def _():
            h_scr[rows, :] = h_scr[rows, :] + acc_scr[rows, :]

        h = h_scr[rows, :]
        mu = jnp.mean(h, axis=1, keepdims=True)
        var = jnp.mean((h - mu) ** 2, axis=1, keepdims=True)
        hn = (h - mu) / jnp.sqrt(var + EPS) * lng_ref[0, 0, :] + lnb_ref[0, 0, :]
        hnh = hn.astype(jnp.bfloat16)
        hn_scr[rows, :] = hnh

        logits = (jnp.dot(hnh, wr_ref[0, :D, :].astype(jnp.bfloat16),
                          preferred_element_type=jnp.float32)
                  + jnp.dot(feat_ref[rows, :].astype(jnp.bfloat16),
                            wr_ref[0, D:, :].astype(jnp.bfloat16),
                            preferred_element_type=jnp.float32)
                  + br_ref[0, 0, :])
        m = jnp.max(logits, axis=1, keepdims=True)
        ex = jnp.exp(logits - m)
        probs = ex / jnp.sum(ex, axis=1, keepdims=True)

        idx = jax.lax.broadcasted_iota(jnp.int32, (TT, E), 1)
        m1 = jnp.max(probs, axis=1, keepdims=True)
        i1 = jnp.min(jnp.where(probs == m1, idx, E), axis=1, keepdims=True)
        oh1 = idx == i1
        probs2 = jnp.where(oh1, -1.0, probs)
        m2 = jnp.max(probs2, axis=1, keepdims=True)
        i2 = jnp.min(jnp.where(probs2 == m2, idx, E), axis=1, keepdims=True)
        oh2 = idx == i2
        denom = m1 + m2 + 1e-9
        g_scr[rows, :] = (jnp.where(oh1, m1, 0.0) + jnp.where(oh2, m2, 0.0)) / denom

    idx = jax.lax.broadcasted_iota(jnp.int32, (TT, E), 1)
    ge = jnp.sum(jnp.where(idx == e, g_scr[rows, :], 0.0),
                 axis=1, keepdims=True)

    def _accumulate(contrib):
        @pl.when(e == 0)
        def _():
            acc_scr[rows, :] = contrib

        @pl.when(e > 0)
        def _():
            acc_scr[rows, :] = acc_scr[rows, :] + contrib

    hnb = hn_scr[rows, :]
    a1 = jnp.dot(hnb, w1_ref[0, 0],
                 preferred_element_type=jnp.float32) + b1_ref[0, 0, :]
    eh = jax.nn.gelu(a1)
    eo = jnp.dot(eh.astype(jnp.bfloat16), w2_ref[0, 0],
                 preferred_element_type=jnp.float32)
    _accumulate(ge * (eo + b2_ref[0, 0, :]))

    @pl.when((s == NS - 1) & (e == E - 1))
    def _():
        out_ref[rows, :] = h_scr[rows, :] + acc_scr[rows, :]


def _run_moe(hid, feat, lng, lnb, wr, br, w1, b1, w2, b2):
    grid = (NC, NS, E, TPC)
    return pl.pallas_call(
        _moe_body,
        grid=grid,
        in_specs=[
            pl.BlockSpec((SPC, D), lambda c, s, e, i: (c, 0)),
            pl.BlockSpec((SPC, NF), lambda c, s, e, i: (c, 0)),
            pl.BlockSpec((1, 1, D), lambda c, s, e, i: (s, 0, 0)),
            pl.BlockSpec((1, 1, D), lambda c, s, e, i: (s, 0, 0)),
            pl.BlockSpec((1, D + NF, E), lambda c, s, e, i: (s, 0, 0)),
            pl.BlockSpec((1, 1, E), lambda c, s, e, i: (s, 0, 0)),
            pl.BlockSpec((1, 1, D, H), lambda c, s, e, i: (s, e, 0, 0)),
            pl.BlockSpec((1, 1, H), lambda c, s, e, i: (s * E + e, 0, 0)),
            pl.BlockSpec((1, 1, H, D), lambda c, s, e, i: (s, e, 0, 0)),
            pl.BlockSpec((1, 1, D), lambda c, s, e, i: (s * E + e, 0, 0)),
        ],
        out_specs=pl.BlockSpec((SPC, D), lambda c, s, e, i: (c, 0)),
        out_shape=jax.ShapeDtypeStruct((S, D), jnp.float32),
        scratch_shapes=[
            pltpu.VMEM((SPC, D), jnp.float32),
            pltpu.VMEM((SPC, D), jnp.bfloat16),
            pltpu.VMEM((SPC, E), jnp.float32),
            pltpu.VMEM((SPC, D), jnp.float32),
        ],
        compiler_params=pltpu.CompilerParams(
            dimension_semantics=("parallel", "arbitrary", "arbitrary",
                                 "arbitrary"),
        ),
    )(hid, feat, lng, lnb, wr, br, w1, b1, w2, b2)


def kernel(hidden, feat,
           ln_g0, ln_b0, Wr0, br0, W1_0, b1_0, W2_0, b2_0,
           ln_g1, ln_b1, Wr1, br1, W1_1, b1_1, W2_1, b2_1,
           ln_g2, ln_b2, Wr2, br2, W1_2, b1_2, W2_2, b2_2):
    lng = jnp.stack([ln_g0, ln_g1, ln_g2]).reshape(NS, 1, D)
    lnb = jnp.stack([ln_b0, ln_b1, ln_b2]).reshape(NS, 1, D)
    wr = jnp.stack([Wr0, Wr1, Wr2])
    br = jnp.stack([br0, br1, br2]).reshape(NS, 1, E)
    w1 = jnp.stack([W1_0, W1_1, W1_2]).astype(jnp.bfloat16)
    b1 = jnp.stack([b1_0, b1_1, b1_2]).reshape(NS * E, 1, H)
    w2 = jnp.stack([W2_0, W2_1, W2_2]).astype(jnp.bfloat16)
    b2 = jnp.stack([b2_0, b2_1, b2_2]).reshape(NS * E, 1, D)
    out = _run_moe(hidden.reshape(S, D), feat.reshape(S, NF),
                   lng, lnb, wr, br, w1, b1, w2, b2)
    return out.reshape(hidden.shape)
